# Initial kernel scaffold; baseline (speedup 1.0000x reference)
import jax
import jax.numpy as jnp
from jax.experimental import pallas as pl


def kernel(entity_emb, basis0, coeff0, self_w0, bias0, basis1, coeff1, self_w1, bias1, gru_Wr, gru_Ur, gru_br, gru_Wz, gru_Uz, gru_bz, gru_Wh, gru_Uh, gru_bh, dec_rel_emb, dec_conv_w, dec_conv_b, dec_fc, dec_fc_b, edge_index, edge_type, triples):
    raise NotImplementedError("write your pallas kernel here")


# speedup vs baseline: 1.2608x; 1.2608x over previous
import functools

import jax
import jax.numpy as jnp
from jax import lax
from jax.experimental import pallas as pl
from jax.experimental.pallas import tpu as pltpu
from jax.experimental.pallas import tpu_sc as plsc

_N = 10000
_R = 24
_D = 256
_E = 160000
_B = 4096
_NB = 30
_NF = 32
_KS = 3
_OL = 2 * _D - _KS + 1

_NC = 2
_NS = 16
_NW = _NC * _NS
_CHUNK = 80
_NCHUNK = 63
_EPW = _CHUNK * _NCHUNK
_EPAD = _EPW * _NW
_TPW = _B // _NW


def _mm_kernel(a_ref, b_ref, o_ref):
    o_ref[...] = jnp.dot(a_ref[...], b_ref[...],
                         preferred_element_type=jnp.float32)


def _rel_weights(coeff, basis_flat):
    return pl.pallas_call(
        _mm_kernel,
        out_shape=jax.ShapeDtypeStruct((_R, _D * _D), jnp.float32),
    )(coeff, basis_flat)


def _xw_kernel(x_ref, w_ref, o_ref):
    o_ref[0] = jnp.dot(x_ref[...], w_ref[0],
                       preferred_element_type=jnp.float32)


def _xw_all(x, w_all):
    bn = 1000
    nb = _N // bn
    return pl.pallas_call(
        _xw_kernel,
        grid=(_R + 1, nb),
        in_specs=[
            pl.BlockSpec((bn, _D), lambda r, i: (i, 0)),
            pl.BlockSpec((1, _D, _D), lambda r, i: (r, 0, 0)),
        ],
        out_specs=pl.BlockSpec((1, bn, _D), lambda r, i: (r, i, 0)),
        out_shape=jax.ShapeDtypeStruct((_R + 1, _N, _D), jnp.float32),
    )(x, w_all)


def _post_kernel(agg_ref, xself_ref, deg_ref, bias_ref, o_ref):
    deg = jnp.maximum(deg_ref[:, 0:1], 1.0)
    o_ref[...] = jnp.maximum(
        agg_ref[...] / deg + xself_ref[...] + bias_ref[...], 0.0)


def _post(agg, xself, deg, bias):
    bn = 1000
    nb = _N // bn
    return pl.pallas_call(
        _post_kernel,
        grid=(nb,),
        in_specs=[
            pl.BlockSpec((bn, _D), lambda i: (i, 0)),
            pl.BlockSpec((bn, _D), lambda i: (i, 0)),
            pl.BlockSpec((bn, _D), lambda i: (i, 0)),
            pl.BlockSpec((1, _D), lambda i: (0, 0)),
        ],
        out_specs=pl.BlockSpec((bn, _D), lambda i: (i, 0)),
        out_shape=jax.ShapeDtypeStruct((_N, _D), jnp.float32),
    )(agg, xself, deg, bias)


def _gru_kernel(x_ref, h_ref, wr_ref, ur_ref, wz_ref, uz_ref, wh_ref, uh_ref,
                br_ref, bz_ref, bh_ref, o_ref):
    x = x_ref[...]
    h = h_ref[...]
    dot = functools.partial(jnp.dot, preferred_element_type=jnp.float32)
    r = jax.nn.sigmoid(dot(x, wr_ref[...]) + dot(h, ur_ref[...]) + br_ref[...])
    z = jax.nn.sigmoid(dot(x, wz_ref[...]) + dot(h, uz_ref[...]) + bz_ref[...])
    ht = jnp.tanh(dot(x, wh_ref[...]) + dot(r * h, uh_ref[...]) + bh_ref[...])
    o_ref[...] = (1.0 - z) * h + z * ht


def _gru(x, h, wr, ur, br, wz, uz, bz, wh, uh, bh):
    bn = 1000
    nb = _N // bn
    blk = pl.BlockSpec((bn, _D), lambda i: (i, 0))
    wspec = pl.BlockSpec((_D, _D), lambda i: (0, 0))
    bspec = pl.BlockSpec((1, _D), lambda i: (0, 0))
    return pl.pallas_call(
        _gru_kernel,
        grid=(nb,),
        in_specs=[blk, blk, wspec, wspec, wspec, wspec, wspec, wspec,
                  bspec, bspec, bspec],
        out_specs=blk,
        out_shape=jax.ShapeDtypeStruct((_N, _D), jnp.float32),
    )(x, h, wr, ur, wz, uz, wh, uh, br.reshape(1, _D), bz.reshape(1, _D),
      bh.reshape(1, _D))


def _dec_kernel(subj_ref, rel_ref, obj_ref, fc_ref, fcb_ref, w_ref, b_ref,
                o_ref):
    comb = jnp.concatenate([subj_ref[...], rel_ref[...]], axis=1)
    c0 = comb[:, 0:_OL]
    c1 = comb[:, 1:_OL + 1]
    c2 = comb[:, 2:_OL + 2]
    acc = jnp.zeros((subj_ref.shape[0], _D), jnp.float32)
    for f in range(_NF):
        cf = w_ref[f, 0] * c0 + w_ref[f, 1] * c1 + w_ref[f, 2] * c2 + b_ref[f]
        cf = jnp.maximum(cf, 0.0)
        acc = acc + jnp.dot(cf, fc_ref[f],
                            preferred_element_type=jnp.float32)
    p = acc + fcb_ref[...]
    o_ref[...] = jnp.sum(p * obj_ref[...], axis=1, keepdims=True)


def _decoder(subj, rel, obj, fc3, fcb, conv_w2, conv_b):
    bb = 256
    nb = _B // bb
    blk = pl.BlockSpec((bb, _D), lambda i: (i, 0))
    out = pl.pallas_call(
        _dec_kernel,
        grid=(nb,),
        in_specs=[
            blk, blk, blk,
            pl.BlockSpec((_NF, _OL, _D), lambda i: (0, 0, 0)),
            pl.BlockSpec((1, _D), lambda i: (0, 0)),
            pl.BlockSpec(memory_space=pltpu.SMEM),
            pl.BlockSpec(memory_space=pltpu.SMEM),
        ],
        out_specs=pl.BlockSpec((bb, 1), lambda i: (i, 0)),
        out_shape=jax.ShapeDtypeStruct((_B, 1), jnp.float32),
    )(subj, rel, obj, fc3, fcb, conv_w2, conv_b)
    return out[:, 0]


def _edge_body(count_deg, xw_hbm, esrc_hbm, etgt_hbm, etype_hbm, ones_hbm,
               agg_ref, deg_ref,
               srcv, tgtv, typev, gidx, rows, onesv, sem):
    c = lax.axis_index("c")
    s = lax.axis_index("s")
    wid = s * _NC + c

    if count_deg:
        pltpu.sync_copy(ones_hbm, onesv)
    base0 = wid * _EPW

    def body(ch, carry):
        base = base0 + ch * _CHUNK
        pltpu.sync_copy(esrc_hbm.at[pl.ds(base, _CHUNK)], srcv)
        pltpu.sync_copy(etgt_hbm.at[pl.ds(base, _CHUNK)], tgtv)
        pltpu.sync_copy(etype_hbm.at[pl.ds(base, _CHUNK)], typev)
        for j in range(_CHUNK // 16):
            sl = pl.ds(j * 16, 16)
            gidx[sl] = typev[sl] * _N + srcv[sl]
        pltpu.async_copy(xw_hbm.at[gidx], rows, sem).wait()
        pltpu.sync_copy(rows, agg_ref.at[tgtv], add=True)
        if count_deg:
            pltpu.sync_copy(onesv, deg_ref.at[tgtv], add=True)
        return carry

    lax.fori_loop(0, _NCHUNK, body, 0)


def _edge_stage(xw_flat, esrc, etgt, etype, count_deg):
    ones = jnp.ones((_CHUNK, _D), jnp.float32)
    agg = jax.new_ref(jnp.zeros((_N + 8, _D), jnp.float32))
    deg = jax.new_ref(jnp.zeros((_N + 8, _D), jnp.float32))
    mesh = plsc.VectorSubcoreMesh(core_axis_name="c", subcore_axis_name="s")
    f = pl.kernel(
        functools.partial(_edge_body, count_deg),
        mesh=mesh,
        out_type=(),
        scratch_types=[
            pltpu.VMEM((_CHUNK,), jnp.int32),
            pltpu.VMEM((_CHUNK,), jnp.int32),
            pltpu.VMEM((_CHUNK,), jnp.int32),
            pltpu.VMEM((_CHUNK,), jnp.int32),
            pltpu.VMEM((_CHUNK, _D), jnp.float32),
            pltpu.VMEM((_CHUNK, _D), jnp.float32),
            pltpu.SemaphoreType.DMA,
        ],
    )
    f(xw_flat, esrc, etgt, etype, ones, agg, deg)
    return jax.freeze(agg)[:_N], jax.freeze(deg)[:_N]


def _dec_gather_body(h_hbm, rel_hbm, ts_hbm, tr_hbm, to_hbm,
                     subj_hbm, relo_hbm, obj_hbm,
                     idxv, rows, sem):
    c = lax.axis_index("c")
    s = lax.axis_index("s")
    base = (s * _NC + c) * _TPW
    pltpu.sync_copy(ts_hbm.at[pl.ds(base, _TPW)], idxv)
    pltpu.async_copy(h_hbm.at[idxv], rows, sem).wait()
    pltpu.sync_copy(rows, subj_hbm.at[pl.ds(base, _TPW)])
    pltpu.sync_copy(tr_hbm.at[pl.ds(base, _TPW)], idxv)
    pltpu.async_copy(rel_hbm.at[idxv], rows, sem).wait()
    pltpu.sync_copy(rows, relo_hbm.at[pl.ds(base, _TPW)])
    pltpu.sync_copy(to_hbm.at[pl.ds(base, _TPW)], idxv)
    pltpu.async_copy(h_hbm.at[idxv], rows, sem).wait()
    pltpu.sync_copy(rows, obj_hbm.at[pl.ds(base, _TPW)])


def _dec_gather(h, rel_emb, ts, tr, to):
    mesh = plsc.VectorSubcoreMesh(core_axis_name="c", subcore_axis_name="s")
    f = pl.kernel(
        _dec_gather_body,
        mesh=mesh,
        out_type=[
            jax.ShapeDtypeStruct((_B, _D), jnp.float32),
            jax.ShapeDtypeStruct((_B, _D), jnp.float32),
            jax.ShapeDtypeStruct((_B, _D), jnp.float32),
        ],
        scratch_types=[
            pltpu.VMEM((_TPW,), jnp.int32),
            pltpu.VMEM((_TPW, _D), jnp.float32),
            pltpu.SemaphoreType.DMA,
        ],
    )
    return f(h, rel_emb, ts, tr, to)


def _layer(x, basis, coeff, self_w, bias, esrc, etgt, etype, deg=None):
    relw = _rel_weights(coeff, basis.reshape(_NB, _D * _D))
    w_all = jnp.concatenate([relw.reshape(_R, _D, _D), self_w[None]], axis=0)
    xw = _xw_all(x, w_all)
    agg, deg_new = _edge_stage(xw[:_R].reshape(_R * _N, _D), esrc, etgt,
                               etype, count_deg=deg is None)
    if deg is None:
        deg = deg_new
    return _post(agg, xw[_R], deg, bias.reshape(1, _D)), deg


def kernel(entity_emb, basis0, coeff0, self_w0, bias0, basis1, coeff1,
           self_w1, bias1, gru_Wr, gru_Ur, gru_br, gru_Wz, gru_Uz, gru_bz,
           gru_Wh, gru_Uh, gru_bh, dec_rel_emb, dec_conv_w, dec_conv_b,
           dec_fc, dec_fc_b, edge_index, edge_type, triples):
    pad = _EPAD - _E
    zpad = jnp.zeros((pad,), jnp.int32)
    esrc = jnp.concatenate([edge_index[0].astype(jnp.int32), zpad])
    etgt = jnp.concatenate([edge_index[1].astype(jnp.int32),
                            jnp.full((pad,), _N, jnp.int32)])
    etype = jnp.concatenate([edge_type.astype(jnp.int32), zpad])

    x1, deg = _layer(entity_emb, basis0, coeff0, self_w0, bias0,
                     esrc, etgt, etype)
    x2, _ = _layer(x1, basis1, coeff1, self_w1, bias1, esrc, etgt, etype,
                   deg=deg)

    h = _gru(x2, entity_emb, gru_Wr, gru_Ur, gru_br, gru_Wz, gru_Uz, gru_bz,
             gru_Wh, gru_Uh, gru_bh)

    subj, rel, obj = _dec_gather(h, dec_rel_emb, triples[:, 0], triples[:, 1],
                                 triples[:, 2])
    fc3 = dec_fc.reshape(_NF, _OL, _D)
    scores = _decoder(subj, rel, obj, fc3, dec_fc_b.reshape(1, _D),
                      dec_conv_w[:, :, 0], dec_conv_b)
    return scores
